# baseline (speedup 1.0000x reference)
import functools

import jax
import jax.numpy as jnp
from jax import lax
from jax.experimental import pallas as pl
from jax.experimental.pallas import tpu as pltpu
from jax.experimental.pallas import tpu_sc as plsc

E = 64
TOPK = 8
ROW_BLK = 512
S1_BLK = 2048
NUM_SC_WORKERS = 32
LANES = 16


def _row_std(sm):
    mu = jnp.mean(sm, axis=1, keepdims=True)
    d = sm - mu
    return jnp.sqrt(jnp.sum(d * d, axis=1, keepdims=True) * (1.0 / (E - 1)))


def _stage1_body(x_ref, wt_ref, b_ref, sm_ref, ssum_ref):
    logits = jnp.dot(x_ref[...], wt_ref[...],
                     preferred_element_type=jnp.float32) + b_ref[...]
    m = jnp.max(logits, axis=1, keepdims=True)
    e = jnp.exp(logits - m)
    sm = e / jnp.sum(e, axis=1, keepdims=True)
    sm_ref[...] = sm
    ssum_ref[0, 0, 0] = jnp.sum(_row_std(sm))


def _stage2_body(n_rows, nsum, sm_ref, thr_ref, ssum0_ref, ssum1_ref,
                 buf_ref, out_ref):
    del buf_ref
    total = ssum0_ref[0, 0, 0]
    for j in range(1, nsum):
        total = total + ssum0_ref[j, 0, 0]
    for j in range(nsum):
        total = total + ssum1_ref[j, 0, 0]
    mean_std = total * (1.0 / n_rows)
    sm = sm_ref[:, :E]
    thr = thr_ref[:, :E]
    alpha = mean_std / (_row_std(sm) + 1e-6)
    mask = sm < thr
    f = jnp.where(mask, jnp.log(sm + 1.0), jnp.exp(sm) - 1.0)
    g = alpha * f
    m = jnp.max(g, axis=1, keepdims=True)
    eg = jnp.exp(g - m)
    out_ref[...] = eg / jnp.sum(eg, axis=1, keepdims=True)


def _sortd(v):
    s, _ = plsc.sort_key_val(v, v, descending=True)
    return s


def _sc_thresh_body(rows_per_tile, sm_hbm, thr_hbm, sm_v, thr_v):
    wid = lax.axis_index("s") * 2 + lax.axis_index("c")
    base = wid * rows_per_tile
    pltpu.sync_copy(sm_hbm.at[pl.ds(base, rows_per_tile)], sm_v)
    iota = lax.broadcasted_iota(jnp.int32, (LANES,), 0)
    lane7 = iota == 7
    neg_inf = jnp.full((LANES,), -jnp.inf, jnp.float32)

    def row_step(r, carry):
        parts = [_sortd(sm_v[r, pl.ds(c * LANES, LANES)]) for c in range(4)]
        t01 = _sortd(jnp.maximum(parts[0], lax.rev(parts[1], (0,))))
        t23 = _sortd(jnp.maximum(parts[2], lax.rev(parts[3], (0,))))
        tt = _sortd(jnp.maximum(t01, lax.rev(t23, (0,))))
        thr = jnp.max(jnp.where(lane7, tt, neg_inf))
        bvec = jnp.full((LANES,), thr, jnp.float32)
        for c in range(4):
            thr_v[r, pl.ds(c * LANES, LANES)] = bvec
        return carry

    lax.fori_loop(0, rows_per_tile, row_step, 0)
    pltpu.sync_copy(thr_v, thr_hbm.at[pl.ds(base, rows_per_tile)])


def _sc_thresholds(sm):
    n = sm.shape[0]
    rows_per_tile = n // NUM_SC_WORKERS
    mesh = plsc.VectorSubcoreMesh(core_axis_name="c", subcore_axis_name="s")
    k = pl.kernel(
        functools.partial(_sc_thresh_body, rows_per_tile),
        out_type=jax.ShapeDtypeStruct((n, E), jnp.float32),
        mesh=mesh,
        scratch_types=[
            pltpu.VMEM((rows_per_tile, E), jnp.float32),
            pltpu.VMEM((rows_per_tile, E), jnp.float32),
        ],
        compiler_params=pltpu.CompilerParams(
            needs_layout_passes=False, use_tc_tiling_on_sc=True),
    )
    return k(sm)


def _stage1_call(x, wt, b2, h, half):
    n, d = x.shape
    ns1 = half // S1_BLK
    return pl.pallas_call(
        _stage1_body,
        grid=(ns1,),
        in_specs=[
            pl.BlockSpec((S1_BLK, d), lambda i: (i + h * ns1, 0)),
            pl.BlockSpec((d, E), lambda i: (0, 0)),
            pl.BlockSpec((1, E), lambda i: (0, 0)),
        ],
        out_specs=[
            pl.BlockSpec((S1_BLK, E), lambda i: (i, 0)),
            pl.BlockSpec((1, 1, 1), lambda i: (i, 0, 0),
                         memory_space=pltpu.SMEM),
        ],
        out_shape=[
            jax.ShapeDtypeStruct((half, E), jnp.float32),
            jax.ShapeDtypeStruct((ns1, 1, 1), jnp.float32),
        ],
        compiler_params=pltpu.CompilerParams(
            dimension_semantics=("arbitrary",)),
    )(x, wt, b2)


def kernel(x, W_gate, b_gate):
    n, d = x.shape
    nblk = n // ROW_BLK
    nh = nblk // 2
    wt = W_gate.T
    b2 = b_gate.reshape(1, E)

    half = n // 2
    sm0, ssum0 = _stage1_call(x, wt, b2, 0, half)
    sm1, ssum1 = _stage1_call(x, wt, b2, 1, half)
    thr0 = _sc_thresholds(sm0)
    thr1 = _sc_thresholds(sm1)

    def stage2_half(h, sm_h, thr_h, buf):
        return pl.pallas_call(
            functools.partial(_stage2_body, n, half // S1_BLK),
            grid=(nh,),
            in_specs=[
                pl.BlockSpec((ROW_BLK, E), lambda i: (i, 0)),
                pl.BlockSpec((ROW_BLK, E), lambda i: (i, 0)),
                pl.BlockSpec(memory_space=pltpu.SMEM),
                pl.BlockSpec(memory_space=pltpu.SMEM),
                pl.BlockSpec(memory_space=pl.ANY),
            ],
            out_specs=pl.BlockSpec((ROW_BLK, E), lambda i: (i + h * nh, 0)),
            out_shape=jax.ShapeDtypeStruct((n, E), jnp.float32),
            input_output_aliases={4: 0},
            compiler_params=pltpu.CompilerParams(
                dimension_semantics=("arbitrary",)),
        )(sm_h, thr_h, ssum0, ssum1, buf)

    buf = jnp.zeros((n, E), jnp.float32)
    buf = stage2_half(0, sm0, thr0, buf)
    gates = stage2_half(1, sm1, thr1, buf)
    return gates


# speedup vs baseline: 1.0886x; 1.0886x over previous
import functools

import jax
import jax.numpy as jnp
from jax import lax
from jax.experimental import pallas as pl
from jax.experimental.pallas import tpu as pltpu
from jax.experimental.pallas import tpu_sc as plsc

E = 64
TOPK = 8
ROW_BLK = 1024
S1_BLK = 1024
NUM_SC_WORKERS = 32
LANES = 16


def _row_std(sm):
    mu = jnp.mean(sm, axis=1, keepdims=True)
    d = sm - mu
    return jnp.sqrt(jnp.sum(d * d, axis=1, keepdims=True) * (1.0 / (E - 1)))


def _stage1_body(x_ref, wt_ref, b_ref, sm_ref, ssum_ref):
    logits = jnp.dot(x_ref[...], wt_ref[...],
                     preferred_element_type=jnp.float32) + b_ref[...]
    m = jnp.max(logits, axis=1, keepdims=True)
    e = jnp.exp(logits - m)
    sm = e / jnp.sum(e, axis=1, keepdims=True)
    sm_ref[...] = sm
    ssum_ref[0, 0, 0] = jnp.sum(_row_std(sm))


def _stage2_body(n_rows, nsum, sm_ref, thr_ref, ssum0_ref, ssum1_ref,
                 buf_ref, out_ref):
    del buf_ref
    total = ssum0_ref[0, 0, 0]
    for j in range(1, nsum):
        total = total + ssum0_ref[j, 0, 0]
    for j in range(nsum):
        total = total + ssum1_ref[j, 0, 0]
    mean_std = total * (1.0 / n_rows)
    sm = sm_ref[:, :E]
    thr = thr_ref[:, :E]
    alpha = mean_std / (_row_std(sm) + 1e-6)
    mask = sm < thr
    f = jnp.where(mask, jnp.log(sm + 1.0), jnp.exp(sm) - 1.0)
    g = alpha * f
    m = jnp.max(g, axis=1, keepdims=True)
    eg = jnp.exp(g - m)
    out_ref[...] = eg / jnp.sum(eg, axis=1, keepdims=True)


def _sortd(v):
    s, _ = plsc.sort_key_val(v, v, descending=True)
    return s


def _sc_thresh_body(rows_per_tile, sm_hbm, thr_hbm, sm_v, thr_v):
    wid = lax.axis_index("s") * 2 + lax.axis_index("c")
    base = wid * rows_per_tile
    pltpu.sync_copy(sm_hbm.at[pl.ds(base, rows_per_tile)], sm_v)
    iota = lax.broadcasted_iota(jnp.int32, (LANES,), 0)
    lane7 = iota == 7
    neg_inf = jnp.full((LANES,), -jnp.inf, jnp.float32)

    def row_step(r, carry):
        parts = [_sortd(sm_v[r, pl.ds(c * LANES, LANES)]) for c in range(4)]
        t01 = _sortd(jnp.maximum(parts[0], lax.rev(parts[1], (0,))))
        t23 = _sortd(jnp.maximum(parts[2], lax.rev(parts[3], (0,))))
        tt = _sortd(jnp.maximum(t01, lax.rev(t23, (0,))))
        thr = jnp.max(jnp.where(lane7, tt, neg_inf))
        bvec = jnp.full((LANES,), thr, jnp.float32)
        for c in range(4):
            thr_v[r, pl.ds(c * LANES, LANES)] = bvec
        return carry

    lax.fori_loop(0, rows_per_tile, row_step, 0)
    pltpu.sync_copy(thr_v, thr_hbm.at[pl.ds(base, rows_per_tile)])


def _sc_thresholds(sm):
    n = sm.shape[0]
    rows_per_tile = n // NUM_SC_WORKERS
    mesh = plsc.VectorSubcoreMesh(core_axis_name="c", subcore_axis_name="s")
    k = pl.kernel(
        functools.partial(_sc_thresh_body, rows_per_tile),
        out_type=jax.ShapeDtypeStruct((n, E), jnp.float32),
        mesh=mesh,
        scratch_types=[
            pltpu.VMEM((rows_per_tile, E), jnp.float32),
            pltpu.VMEM((rows_per_tile, E), jnp.float32),
        ],
        compiler_params=pltpu.CompilerParams(
            needs_layout_passes=False, use_tc_tiling_on_sc=True),
    )
    return k(sm)


def _stage1_call(x, wt, b2, h, half):
    n, d = x.shape
    ns1 = half // S1_BLK
    return pl.pallas_call(
        _stage1_body,
        grid=(ns1,),
        in_specs=[
            pl.BlockSpec((S1_BLK, d), lambda i: (i + h * ns1, 0)),
            pl.BlockSpec((d, E), lambda i: (0, 0)),
            pl.BlockSpec((1, E), lambda i: (0, 0)),
        ],
        out_specs=[
            pl.BlockSpec((S1_BLK, E), lambda i: (i, 0)),
            pl.BlockSpec((1, 1, 1), lambda i: (i, 0, 0),
                         memory_space=pltpu.SMEM),
        ],
        out_shape=[
            jax.ShapeDtypeStruct((half, E), jnp.float32),
            jax.ShapeDtypeStruct((ns1, 1, 1), jnp.float32),
        ],
        compiler_params=pltpu.CompilerParams(
            dimension_semantics=("arbitrary",)),
    )(x, wt, b2)


def kernel(x, W_gate, b_gate):
    n, d = x.shape
    nblk = n // ROW_BLK
    nh = nblk // 2
    wt = W_gate.T
    b2 = b_gate.reshape(1, E)

    half = n // 2
    sm0, ssum0 = _stage1_call(x, wt, b2, 0, half)
    sm1, ssum1 = _stage1_call(x, wt, b2, 1, half)
    thr0 = _sc_thresholds(sm0)
    thr1 = _sc_thresholds(sm1)

    def stage2_half(h, sm_h, thr_h, buf):
        return pl.pallas_call(
            functools.partial(_stage2_body, n, half // S1_BLK),
            grid=(nh,),
            in_specs=[
                pl.BlockSpec((ROW_BLK, E), lambda i: (i, 0)),
                pl.BlockSpec((ROW_BLK, E), lambda i: (i, 0)),
                pl.BlockSpec(memory_space=pltpu.SMEM),
                pl.BlockSpec(memory_space=pltpu.SMEM),
                pl.BlockSpec(memory_space=pl.ANY),
            ],
            out_specs=pl.BlockSpec((ROW_BLK, E), lambda i: (i + h * nh, 0)),
            out_shape=jax.ShapeDtypeStruct((n, E), jnp.float32),
            input_output_aliases={4: 0},
            compiler_params=pltpu.CompilerParams(
                dimension_semantics=("arbitrary",)),
        )(sm_h, thr_h, ssum0, ssum1, buf)

    buf = jnp.zeros((n, E), jnp.float32)
    buf = stage2_half(0, sm0, thr0, buf)
    gates = stage2_half(1, sm1, thr1, buf)
    return gates
